# baseline (speedup 1.0000x reference)
import functools

import jax
import jax.numpy as jnp
from jax import lax
from jax.experimental import pallas as pl
from jax.experimental.pallas import tpu as pltpu

_EPS = 1e-12
_ROWS = 68229
_ROWS_PAD = 68608
_M = _ROWS_PAD * 8 // 128
_BLK = 536
_TC_GRID = _M // _BLK


def _tc_loss(xs_ref, ts_ref, out_ref, acc_ref):
    i = pl.program_id(0)

    @pl.when(i == 0)
    def _init():
        acc_ref[0] = 0.0
        acc_ref[1] = 0.0

    xb = xs_ref[...]
    tb = ts_ref[...]

    cls = lax.broadcasted_iota(jnp.int32, (_BLK, 128), 1) % 8

    obj01 = jnp.where((tb > 0.0) & (cls == 4), 1.0, 0.0)
    li = lax.broadcasted_iota(jnp.int32, (128, 128), 0)
    lj = lax.broadcasted_iota(jnp.int32, (128, 128), 1)
    bmat = jnp.where((li // 8 == lj // 8) & (li % 8 == 4), 1.0, 0.0)
    objb = jnp.dot(obj01, bmat, preferred_element_type=jnp.float32)

    p = jnp.clip(xb, _EPS, 1.0 - _EPS)
    elem = -(tb * jnp.log(p) + (1.0 - tb) * jnp.log(1.0 - p))
    take = (objb > 0.0) & (cls < 4)
    acc_ref[0] += jnp.sum(jnp.where(take, elem, 0.0))
    acc_ref[1] += jnp.sum(obj01)

    @pl.when(i == _TC_GRID - 1)
    def _fin():
        denom = jnp.maximum(acc_ref[1] * 2.0, 1.0)
        out_ref[...] = jnp.full((1, 1), acc_ref[0] / denom, jnp.float32)


def _dense_view(a):
    rows, _ = a.shape
    sl = a[:, 0:8]
    padded = jnp.pad(sl, ((0, _ROWS_PAD - rows), (0, 0)))
    return padded.reshape(_M, 128)


def kernel(x, target):
    b, n, c = x.shape
    xi = _dense_view(x.reshape(b * n, c))
    ti = _dense_view(target.reshape(b * n, c))

    spec = pl.BlockSpec((_BLK, 128), lambda i: (i, 0))
    out = pl.pallas_call(
        _tc_loss,
        grid=(_TC_GRID,),
        in_specs=[spec, spec],
        out_specs=pl.BlockSpec((1, 1), lambda i: (0, 0)),
        out_shape=jax.ShapeDtypeStruct((1, 1), jnp.float32),
        scratch_shapes=[pltpu.SMEM((2,), jnp.float32)],
        compiler_params=pltpu.CompilerParams(
            allow_input_fusion=[True, True]),
    )(xi, ti)
    return out[0, 0]


# speedup vs baseline: 13.9798x; 13.9798x over previous
import functools

import jax
import jax.numpy as jnp
from jax.experimental import pallas as pl
from jax.experimental.pallas import tpu as pltpu

_EPS = 1e-12
_LANES = 128
_BLK = 136


def _loss_kernel(x0, x1, x2, x3, t0, t1, t2, t3, t4, out_ref, acc_ref,
                 *, n_blocks):
    i = pl.program_id(0)

    @pl.when(i == 0)
    def _init():
        acc_ref[0] = 0.0
        acc_ref[1] = 0.0

    obj = t4[...] > 0.0

    def bce(x_ref, t_ref):
        p = jnp.clip(x_ref[...], _EPS, 1.0 - _EPS)
        t = t_ref[...]
        return -(t * jnp.log(p) + (1.0 - t) * jnp.log(1.0 - p))

    elem = bce(x0, t0) + bce(x1, t1) + bce(x2, t2) + bce(x3, t3)
    acc_ref[0] += jnp.sum(jnp.where(obj, elem, 0.0))
    acc_ref[1] += jnp.sum(jnp.where(obj, 1.0, 0.0))

    @pl.when(i == n_blocks - 1)
    def _finalize():
        denom = jnp.maximum(acc_ref[1] * 2.0, 1.0)
        out_ref[...] = jnp.full((1, 1), acc_ref[0] / denom, jnp.float32)


def kernel(x, target):
    b, n, c = x.shape
    rows = b * n
    n_blocks = pl.cdiv(rows, _BLK * _LANES)
    padded = n_blocks * _BLK * _LANES
    pad = jnp.zeros((padded - rows,), jnp.float32)

    def plane(a, ch):
        return jnp.concatenate([a[:, :, ch].reshape(-1), pad]).reshape(
            n_blocks * _BLK, _LANES)

    planes = [plane(x, ch) for ch in range(4)]
    planes += [plane(target, ch) for ch in range(5)]

    spec = pl.BlockSpec((_BLK, _LANES), lambda i: (i, 0))
    out = pl.pallas_call(
        functools.partial(_loss_kernel, n_blocks=n_blocks),
        grid=(n_blocks,),
        in_specs=[spec] * 9,
        out_specs=pl.BlockSpec((1, 1), lambda i: (0, 0)),
        out_shape=jax.ShapeDtypeStruct((1, 1), jnp.float32),
        scratch_shapes=[pltpu.SMEM((2,), jnp.float32)],
    )(*planes)
    return out[0, 0]


# speedup vs baseline: 14.0146x; 1.0025x over previous
import functools

import jax
import jax.numpy as jnp
from jax.experimental import pallas as pl
from jax.experimental.pallas import tpu as pltpu

_EPS = 1e-12
_LANES = 128
_BLK = 136


def _loss_kernel(x0, x1, x2, x3, t0, t1, t2, t3, t4, out_ref, acc_ref,
                 *, n_blocks):
    i = pl.program_id(0)

    @pl.when(i == 0)
    def _init():
        acc_ref[0] = 0.0
        acc_ref[1] = 0.0

    obj = t4[...] > 0.0

    def bce(x_ref, t_ref):
        p = jnp.clip(x_ref[...], _EPS, 1.0 - _EPS)
        t = t_ref[...]
        return -(t * jnp.log(p) + (1.0 - t) * jnp.log(1.0 - p))

    elem = bce(x0, t0) + bce(x1, t1) + bce(x2, t2) + bce(x3, t3)
    acc_ref[0] += jnp.sum(jnp.where(obj, elem, 0.0))
    acc_ref[1] += jnp.sum(jnp.where(obj, 1.0, 0.0))

    @pl.when(i == n_blocks - 1)
    def _finalize():
        denom = jnp.maximum(acc_ref[1] * 2.0, 1.0)
        out_ref[...] = jnp.full((1, 1), acc_ref[0] / denom, jnp.float32)


def kernel(x, target):
    b, n, c = x.shape
    rows = b * n
    n_blocks = pl.cdiv(rows, _BLK * _LANES)
    padded = n_blocks * _BLK * _LANES
    pad = jnp.zeros((padded - rows,), jnp.float32)

    def plane(a, ch):
        p = jnp.max(a[:, :, ch:ch + 1], axis=-1).reshape(-1)
        return jnp.concatenate([p, pad]).reshape(n_blocks * _BLK, _LANES)

    planes = [plane(x, ch) for ch in range(4)]
    planes += [plane(target, ch) for ch in range(5)]

    spec = pl.BlockSpec((_BLK, _LANES), lambda i: (i, 0))
    out = pl.pallas_call(
        functools.partial(_loss_kernel, n_blocks=n_blocks),
        grid=(n_blocks,),
        in_specs=[spec] * 9,
        out_specs=pl.BlockSpec((1, 1), lambda i: (0, 0)),
        out_shape=jax.ShapeDtypeStruct((1, 1), jnp.float32),
        scratch_shapes=[pltpu.SMEM((2,), jnp.float32)],
    )(*planes)
    return out[0, 0]


# speedup vs baseline: 14.0203x; 1.0004x over previous
import functools

import jax
import jax.numpy as jnp
from jax.experimental import pallas as pl
from jax.experimental.pallas import tpu as pltpu

_EPS = 1e-12
_LANES = 128
_BLK = 136


def _loss_kernel(x0, x1, x2, x3, t0, t1, t2, t3, t4, out_ref, acc_ref,
                 *, n_blocks):
    i = pl.program_id(0)

    @pl.when(i == 0)
    def _init():
        acc_ref[0] = 0.0
        acc_ref[1] = 0.0

    obj = t4[...] > 0.0

    def bce(x_ref, t_ref):
        p = jnp.clip(x_ref[...], _EPS, 1.0 - _EPS)
        t = t_ref[...]
        return -(t * jnp.log(p) + (1.0 - t) * jnp.log(1.0 - p))

    elem = bce(x0, t0) + bce(x1, t1) + bce(x2, t2) + bce(x3, t3)
    acc_ref[0] += jnp.sum(jnp.where(obj, elem, 0.0))
    acc_ref[1] += jnp.sum(jnp.where(obj, 1.0, 0.0))

    @pl.when(i == n_blocks - 1)
    def _finalize():
        denom = jnp.maximum(acc_ref[1] * 2.0, 1.0)
        out_ref[...] = jnp.full((1, 1), acc_ref[0] / denom, jnp.float32)


def kernel(x, target):
    b, n, c = x.shape
    rows = b * n
    n_blocks = pl.cdiv(rows, _BLK * _LANES)
    padded = n_blocks * _BLK * _LANES
    pad = jnp.zeros((padded - rows,), jnp.float32)

    def plane(a, ch):
        return jnp.concatenate([a[:, :, ch].reshape(-1), pad]).reshape(
            n_blocks * _BLK, _LANES)

    planes = [plane(x, ch) for ch in range(4)]
    planes += [plane(target, ch) for ch in range(5)]

    spec = pl.BlockSpec((_BLK, _LANES), lambda i: (i, 0))
    out = pl.pallas_call(
        functools.partial(_loss_kernel, n_blocks=n_blocks),
        grid=(n_blocks,),
        in_specs=[spec] * 9,
        out_specs=pl.BlockSpec((1, 1), lambda i: (0, 0)),
        out_shape=jax.ShapeDtypeStruct((1, 1), jnp.float32),
        scratch_shapes=[pltpu.SMEM((2,), jnp.float32)],
    )(*planes)
    return out[0, 0]


# speedup vs baseline: 18.0005x; 1.2839x over previous
import functools

import jax
import jax.numpy as jnp
from jax.experimental import pallas as pl
from jax.experimental.pallas import tpu as pltpu

_EPS = 1e-12
_LANES = 128
_BLK = 136


def _loss_kernel(x0, x1, x2, x3, t0, t1, t2, t3, t4, out_ref, acc_ref,
                 *, n_blocks):
    i = pl.program_id(0)

    @pl.when(i == 0)
    def _init():
        acc_ref[0] = 0.0
        acc_ref[1] = 0.0

    obj = t4[...] > 0.0

    def bce(x_ref, t_ref):
        p = jnp.clip(x_ref[...], _EPS, 1.0 - _EPS)
        t = t_ref[...]
        return -(t * jnp.log(p) + (1.0 - t) * jnp.log(1.0 - p))

    elem = bce(x0, t0) + bce(x1, t1) + bce(x2, t2) + bce(x3, t3)
    acc_ref[0] += jnp.sum(jnp.where(obj, elem, 0.0))
    acc_ref[1] += jnp.sum(jnp.where(obj, 1.0, 0.0))

    @pl.when(i == n_blocks - 1)
    def _finalize():
        denom = jnp.maximum(acc_ref[1] * 2.0, 1.0)
        out_ref[...] = jnp.full((1, 1), acc_ref[0] / denom, jnp.float32)


def kernel(x, target):
    b, n, c = x.shape
    rows = b * n
    n_blocks = pl.cdiv(rows, _BLK * _LANES)
    padded = n_blocks * _BLK * _LANES
    pad = jnp.zeros((padded - rows,), jnp.float32)

    def plane(a, ch):
        return jnp.concatenate([a[:, :, ch].reshape(-1), pad]).reshape(
            n_blocks * _BLK, _LANES)

    planes = [plane(x, ch) for ch in range(4)]
    planes += [plane(target, ch) for ch in range(5)]

    spec = pl.BlockSpec((_BLK, _LANES), lambda i: (i, 0))
    out = pl.pallas_call(
        functools.partial(_loss_kernel, n_blocks=n_blocks),
        grid=(n_blocks,),
        in_specs=[spec] * 9,
        out_specs=pl.BlockSpec((1, 1), lambda i: (0, 0)),
        out_shape=jax.ShapeDtypeStruct((1, 1), jnp.float32),
        scratch_shapes=[pltpu.SMEM((2,), jnp.float32)],
        compiler_params=pltpu.CompilerParams(
            allow_input_fusion=[True] * 9),
    )(*planes)
    return out[0, 0]
